# baseline (speedup 1.0000x reference)
import functools

import jax
import jax.numpy as jnp
from jax import lax
from jax.experimental import pallas as pl
from jax.experimental.pallas import tpu as pltpu
from jax.experimental.pallas import tpu_sc as plsc

N = 65536
NNZ = 4194304
IN_DIM = 512
OUT_DIM = 512

NC = 2
NS = 16
NW = NC * NS
EPT = NNZ // NW
CH = 2048
NCHUNK = EPT // CH
NPAIR = NCHUNK // 2
L = 16


def _mv_in_body(w_ref, x_ref, b_ref, o_ref):
    acc = jnp.dot(w_ref[...], x_ref[...], preferred_element_type=jnp.float32)
    o_ref[...] = jnp.maximum(acc + b_ref[...], 0.0)


BM = 4096


def _h_matvec(W_in, x, b_in):
    grid = N // BM
    return pl.pallas_call(
        _mv_in_body,
        grid=(grid,),
        in_specs=[
            pl.BlockSpec((BM, IN_DIM), lambda i: (i, 0)),
            pl.BlockSpec((IN_DIM, 1), lambda i: (0, 0)),
            pl.BlockSpec((BM, 1), lambda i: (i, 0)),
        ],
        out_specs=pl.BlockSpec((BM, 1), lambda i: (i, 0)),
        out_shape=jax.ShapeDtypeStruct((N, 1), jnp.float32),
    )(W_in, x.reshape(IN_DIM, 1), b_in.reshape(N, 1))


BK = 4096


def _mv_out_body(w_ref, y0_ref, y1_ref, b_ref, o_ref):
    i = pl.program_id(0)
    v = jnp.maximum(y0_ref[0, 0, :] + y1_ref[0, 0, :], 0.0).reshape(BK, 1)
    part = jnp.dot(w_ref[...], v, preferred_element_type=jnp.float32)

    @pl.when(i == 0)
    def _():
        o_ref[...] = b_ref[...] + part

    @pl.when(i > 0)
    def _():
        o_ref[...] += part


def _out_matvec(W_out, y_parts, b_out):
    grid = N // BK
    yp3 = y_parts.reshape(NC, 1, N)
    return pl.pallas_call(
        _mv_out_body,
        grid=(grid,),
        in_specs=[
            pl.BlockSpec((OUT_DIM, BK), lambda i: (0, i)),
            pl.BlockSpec((1, 1, BK), lambda i: (0, 0, i)),
            pl.BlockSpec((1, 1, BK), lambda i: (1, 0, i)),
            pl.BlockSpec((OUT_DIM, 1), lambda i: (0, 0)),
        ],
        out_specs=pl.BlockSpec((OUT_DIM, 1), lambda i: (0, 0)),
        out_shape=jax.ShapeDtypeStruct((OUT_DIM, 1), jnp.float32),
    )(W_out, yp3, yp3, b_out.reshape(OUT_DIM, 1))


NRED = 2
NR = N // NRED
SEG = NR // NS


def _sc_edge_body(h_hbm, cols_hbm, w_hbm, rows_hbm, yp_hbm, c_hbm,
                  hy_v, ia_v, ib_v, va_v, vb_v, ca_v, cb_v, acc_v, tmp_v,
                  h_sp, y_sp,
                  sia, sib, sva, svb, sca, scb):
    core = lax.axis_index("c")
    sid = lax.axis_index("s")
    wid = sid * NC + core
    base = wid * EPT

    def start_in(src, ci, buf, sem):
        pltpu.async_copy(src.at[pl.ds(base + ci * CH, CH)], buf, sem)

    def wait_in(src, buf, sem):
        pltpu.make_async_copy(src.at[pl.ds(base, CH)], buf, sem).wait()

    def start_out(buf, ci, sem):
        pltpu.async_copy(buf, c_hbm.at[pl.ds(base + ci * CH, CH)], sem)

    def wait_out(buf, sem):
        pltpu.make_async_copy(buf, c_hbm.at[pl.ds(base, CH)], sem).wait()

    @pl.when(sid == 0)
    def _():
        pltpu.sync_copy(h_hbm, h_sp)

    plsc.subcore_barrier()
    pltpu.sync_copy(h_sp, hy_v)
    start_in(cols_hbm, 0, ia_v, sia)
    start_in(w_hbm, 0, va_v, sva)

    def compute_a(idx_v, w_v, c_v):
        @plsc.parallel_loop(0, CH // L, unroll=8)
        def _(j):
            s = pl.ds(j * L, L)
            c_v[s] = plsc.load_gather(hy_v, [idx_v[s]]) * w_v[s]

    def pair_a(p, _):
        even = 2 * p
        start_in(cols_hbm, even + 1, ib_v, sib)
        start_in(w_hbm, even + 1, vb_v, svb)
        wait_in(cols_hbm, ia_v, sia)
        wait_in(w_hbm, va_v, sva)

        @pl.when(p > 0)
        def _():
            wait_out(ca_v, sca)

        compute_a(ia_v, va_v, ca_v)
        start_out(ca_v, even, sca)

        @pl.when(p < NPAIR - 1)
        def _():
            start_in(cols_hbm, even + 2, ia_v, sia)
            start_in(w_hbm, even + 2, va_v, sva)

        wait_in(cols_hbm, ib_v, sib)
        wait_in(w_hbm, vb_v, svb)

        @pl.when(p > 0)
        def _():
            wait_out(cb_v, scb)

        compute_a(ib_v, vb_v, cb_v)
        start_out(cb_v, even + 1, scb)
        return 0

    lax.fori_loop(0, NPAIR, pair_a, 0)
    wait_out(ca_v, sca)
    wait_out(cb_v, scb)

    zeros = jnp.zeros((L,), jnp.float32)

    @plsc.parallel_loop(0, N // L, unroll=8)
    def _(i):
        hy_v[pl.ds(i * L, L)] = zeros

    start_in(rows_hbm, 0, ia_v, sia)
    start_in(c_hbm, 0, ca_v, sca)

    def compute_b(idx_v, c_v):
        def vec_b(j, _):
            s = pl.ds(j * L, L)
            plsc.addupdate_scatter(hy_v, [idx_v[s]], c_v[s])
            return 0

        lax.fori_loop(0, CH // L, vec_b, 0, unroll=8)

    def pair_b(p, _):
        even = 2 * p
        start_in(rows_hbm, even + 1, ib_v, sib)
        start_in(c_hbm, even + 1, cb_v, scb)
        wait_in(rows_hbm, ia_v, sia)
        wait_in(c_hbm, ca_v, sca)
        compute_b(ia_v, ca_v)

        @pl.when(p < NPAIR - 1)
        def _():
            start_in(rows_hbm, even + 2, ia_v, sia)
            start_in(c_hbm, even + 2, ca_v, sca)

        wait_in(rows_hbm, ib_v, sib)
        wait_in(c_hbm, cb_v, scb)
        compute_b(ib_v, cb_v)
        return 0

    lax.fori_loop(0, NPAIR, pair_b, 0)

    lo = sid * SEG
    for r in range(NRED):
        pltpu.sync_copy(hy_v.at[pl.ds(r * NR, NR)], y_sp.at[sid])
        plsc.subcore_barrier()
        pltpu.sync_copy(y_sp.at[0, pl.ds(lo, SEG)], acc_v)
        for k in range(1, NS):
            pltpu.sync_copy(y_sp.at[k, pl.ds(lo, SEG)], tmp_v)

            @plsc.parallel_loop(0, SEG // L, unroll=8)
            def _(i):
                s = pl.ds(i * L, L)
                acc_v[s] = acc_v[s] + tmp_v[s]

        pltpu.sync_copy(acc_v, yp_hbm.at[core, pl.ds(r * NR + lo, SEG)])
        plsc.subcore_barrier()


_sc_edges = functools.partial(
    pl.kernel,
    out_type=(
        jax.ShapeDtypeStruct((NC, N), jnp.float32),
        jax.ShapeDtypeStruct((NNZ,), jnp.float32),
    ),
    mesh=plsc.VectorSubcoreMesh(
        core_axis_name="c", subcore_axis_name="s",
        num_cores=NC, num_subcores=NS,
    ),
    scratch_types=[
        pltpu.VMEM((N,), jnp.float32),
        pltpu.VMEM((CH,), jnp.int32),
        pltpu.VMEM((CH,), jnp.int32),
        pltpu.VMEM((CH,), jnp.float32),
        pltpu.VMEM((CH,), jnp.float32),
        pltpu.VMEM((CH,), jnp.float32),
        pltpu.VMEM((CH,), jnp.float32),
        pltpu.VMEM((SEG,), jnp.float32),
        pltpu.VMEM((SEG,), jnp.float32),
        pltpu.VMEM_SHARED((N,), jnp.float32),
        pltpu.VMEM_SHARED((NS, NR), jnp.float32),
        pltpu.SemaphoreType.DMA,
        pltpu.SemaphoreType.DMA,
        pltpu.SemaphoreType.DMA,
        pltpu.SemaphoreType.DMA,
        pltpu.SemaphoreType.DMA,
        pltpu.SemaphoreType.DMA,
    ],
    compiler_params=pltpu.CompilerParams(needs_layout_passes=False),
)(_sc_edge_body)


def kernel(x, rows, cols, adj_weights, W_in, b_in, W_out, b_out):
    h = _h_matvec(W_in, x, b_in)
    y_parts, _ = _sc_edges(h.reshape(N), cols, adj_weights, rows)
    out = _out_matvec(W_out, y_parts, b_out)
    return out.reshape(OUT_DIM)


# speedup vs baseline: 1.0413x; 1.0413x over previous
import functools

import jax
import jax.numpy as jnp
from jax import lax
from jax.experimental import pallas as pl
from jax.experimental.pallas import tpu as pltpu
from jax.experimental.pallas import tpu_sc as plsc

N = 65536
NNZ = 4194304
IN_DIM = 512
OUT_DIM = 512

NC = 2
NS = 16
NW = NC * NS
EPT = NNZ // NW
CH = 4096
NCHUNK = EPT // CH
NPAIR = NCHUNK // 2
L = 16


def _mv_in_body(w_ref, x_ref, b_ref, o_ref):
    acc = jnp.dot(w_ref[...], x_ref[...], preferred_element_type=jnp.float32)
    o_ref[...] = jnp.maximum(acc + b_ref[...], 0.0)


BM = 4096


def _h_matvec(W_in, x, b_in):
    grid = N // BM
    return pl.pallas_call(
        _mv_in_body,
        grid=(grid,),
        in_specs=[
            pl.BlockSpec((BM, IN_DIM), lambda i: (i, 0)),
            pl.BlockSpec((IN_DIM, 1), lambda i: (0, 0)),
            pl.BlockSpec((BM, 1), lambda i: (i, 0)),
        ],
        out_specs=pl.BlockSpec((BM, 1), lambda i: (i, 0)),
        out_shape=jax.ShapeDtypeStruct((N, 1), jnp.float32),
    )(W_in, x.reshape(IN_DIM, 1), b_in.reshape(N, 1))


BK = 4096


def _mv_out_body(w_ref, y0_ref, y1_ref, b_ref, o_ref):
    i = pl.program_id(0)
    v = jnp.maximum(y0_ref[0, 0, :] + y1_ref[0, 0, :], 0.0).reshape(BK, 1)
    part = jnp.dot(w_ref[...], v, preferred_element_type=jnp.float32)

    @pl.when(i == 0)
    def _():
        o_ref[...] = b_ref[...] + part

    @pl.when(i > 0)
    def _():
        o_ref[...] += part


def _out_matvec(W_out, y_parts, b_out):
    grid = N // BK
    yp3 = y_parts.reshape(NC, 1, N)
    return pl.pallas_call(
        _mv_out_body,
        grid=(grid,),
        in_specs=[
            pl.BlockSpec((OUT_DIM, BK), lambda i: (0, i)),
            pl.BlockSpec((1, 1, BK), lambda i: (0, 0, i)),
            pl.BlockSpec((1, 1, BK), lambda i: (1, 0, i)),
            pl.BlockSpec((OUT_DIM, 1), lambda i: (0, 0)),
        ],
        out_specs=pl.BlockSpec((OUT_DIM, 1), lambda i: (0, 0)),
        out_shape=jax.ShapeDtypeStruct((OUT_DIM, 1), jnp.float32),
    )(W_out, yp3, yp3, b_out.reshape(OUT_DIM, 1))


NRED = 2
NR = N // NRED
SEG = NR // NS


def _sc_edge_body(h_hbm, cols_hbm, w_hbm, rows_hbm, yp_hbm, c_hbm,
                  hy_v, ia_v, ib_v, va_v, vb_v, ca_v, cb_v,
                  h_sp, y_sp,
                  sia, sib, sva, svb, sca, scb):
    core = lax.axis_index("c")
    sid = lax.axis_index("s")
    wid = sid * NC + core
    base = wid * EPT

    def start_in(src, ci, buf, sem):
        pltpu.async_copy(src.at[pl.ds(base + ci * CH, CH)], buf, sem)

    def wait_in(src, buf, sem):
        pltpu.make_async_copy(src.at[pl.ds(base, CH)], buf, sem).wait()

    def start_out(buf, ci, sem):
        pltpu.async_copy(buf, c_hbm.at[pl.ds(base + ci * CH, CH)], sem)

    def wait_out(buf, sem):
        pltpu.make_async_copy(buf, c_hbm.at[pl.ds(base, CH)], sem).wait()

    @pl.when(sid == 0)
    def _():
        pltpu.sync_copy(h_hbm, h_sp)

    plsc.subcore_barrier()
    pltpu.sync_copy(h_sp, hy_v)
    start_in(cols_hbm, 0, ia_v, sia)
    start_in(w_hbm, 0, va_v, sva)

    def compute_a(idx_v, w_v, c_v):
        @plsc.parallel_loop(0, CH // L, unroll=8)
        def _(j):
            s = pl.ds(j * L, L)
            c_v[s] = plsc.load_gather(hy_v, [idx_v[s]]) * w_v[s]

    def pair_a(p, _):
        even = 2 * p
        start_in(cols_hbm, even + 1, ib_v, sib)
        start_in(w_hbm, even + 1, vb_v, svb)
        wait_in(cols_hbm, ia_v, sia)
        wait_in(w_hbm, va_v, sva)

        @pl.when(p > 0)
        def _():
            wait_out(ca_v, sca)

        compute_a(ia_v, va_v, ca_v)
        start_out(ca_v, even, sca)

        @pl.when(p < NPAIR - 1)
        def _():
            start_in(cols_hbm, even + 2, ia_v, sia)
            start_in(w_hbm, even + 2, va_v, sva)

        wait_in(cols_hbm, ib_v, sib)
        wait_in(w_hbm, vb_v, svb)

        @pl.when(p > 0)
        def _():
            wait_out(cb_v, scb)

        compute_a(ib_v, vb_v, cb_v)
        start_out(cb_v, even + 1, scb)
        return 0

    lax.fori_loop(0, NPAIR, pair_a, 0)
    wait_out(ca_v, sca)
    wait_out(cb_v, scb)

    zeros = jnp.zeros((L,), jnp.float32)

    @plsc.parallel_loop(0, N // L, unroll=8)
    def _(i):
        hy_v[pl.ds(i * L, L)] = zeros

    start_in(rows_hbm, 0, ia_v, sia)
    start_in(c_hbm, 0, ca_v, sca)

    def compute_b(idx_v, c_v):
        def vec_b(j, _):
            s = pl.ds(j * L, L)
            plsc.addupdate_scatter(hy_v, [idx_v[s]], c_v[s])
            return 0

        lax.fori_loop(0, CH // L, vec_b, 0, unroll=8)

    def pair_b(p, _):
        even = 2 * p
        start_in(rows_hbm, even + 1, ib_v, sib)
        start_in(c_hbm, even + 1, cb_v, scb)
        wait_in(rows_hbm, ia_v, sia)
        wait_in(c_hbm, ca_v, sca)
        compute_b(ia_v, ca_v)

        @pl.when(p < NPAIR - 1)
        def _():
            start_in(rows_hbm, even + 2, ia_v, sia)
            start_in(c_hbm, even + 2, ca_v, sca)

        wait_in(rows_hbm, ib_v, sib)
        wait_in(c_hbm, cb_v, scb)
        compute_b(ib_v, cb_v)
        return 0

    lax.fori_loop(0, NPAIR, pair_b, 0)

    lo = sid * SEG
    for r in range(NRED):
        pltpu.sync_copy(hy_v.at[pl.ds(r * NR, NR)], y_sp.at[sid])
        plsc.subcore_barrier()
        pltpu.sync_copy(y_sp.at[0, pl.ds(lo, SEG)], va_v.at[pl.ds(0, SEG)])
        for k in range(1, NS):
            pltpu.sync_copy(y_sp.at[k, pl.ds(lo, SEG)], vb_v.at[pl.ds(0, SEG)])

            @plsc.parallel_loop(0, SEG // L, unroll=8)
            def _(i):
                s = pl.ds(i * L, L)
                va_v[s] = va_v[s] + vb_v[s]

        pltpu.sync_copy(va_v.at[pl.ds(0, SEG)],
                        yp_hbm.at[core, pl.ds(r * NR + lo, SEG)])
        plsc.subcore_barrier()


_sc_edges = functools.partial(
    pl.kernel,
    out_type=(
        jax.ShapeDtypeStruct((NC, N), jnp.float32),
        jax.ShapeDtypeStruct((NNZ,), jnp.float32),
    ),
    mesh=plsc.VectorSubcoreMesh(
        core_axis_name="c", subcore_axis_name="s",
        num_cores=NC, num_subcores=NS,
    ),
    scratch_types=[
        pltpu.VMEM((N,), jnp.float32),
        pltpu.VMEM((CH,), jnp.int32),
        pltpu.VMEM((CH,), jnp.int32),
        pltpu.VMEM((CH,), jnp.float32),
        pltpu.VMEM((CH,), jnp.float32),
        pltpu.VMEM((CH,), jnp.float32),
        pltpu.VMEM((CH,), jnp.float32),
        pltpu.VMEM_SHARED((N,), jnp.float32),
        pltpu.VMEM_SHARED((NS, NR), jnp.float32),
        pltpu.SemaphoreType.DMA,
        pltpu.SemaphoreType.DMA,
        pltpu.SemaphoreType.DMA,
        pltpu.SemaphoreType.DMA,
        pltpu.SemaphoreType.DMA,
        pltpu.SemaphoreType.DMA,
    ],
    compiler_params=pltpu.CompilerParams(needs_layout_passes=False),
)(_sc_edge_body)


def kernel(x, rows, cols, adj_weights, W_in, b_in, W_out, b_out):
    h = _h_matvec(W_in, x, b_in)
    y_parts, _ = _sc_edges(h.reshape(N), cols, adj_weights, rows)
    out = _out_matvec(W_out, y_parts, b_out)
    return out.reshape(OUT_DIM)


# speedup vs baseline: 1.1201x; 1.0757x over previous
import functools

import jax
import jax.numpy as jnp
from jax import lax
from jax.experimental import pallas as pl
from jax.experimental.pallas import tpu as pltpu
from jax.experimental.pallas import tpu_sc as plsc

N = 65536
NNZ = 4194304
IN_DIM = 512
OUT_DIM = 512

NC = 2
NS = 16
NW = NC * NS
EPT = NNZ // NW
CH = 8192
NCHUNK = EPT // CH
NPAIR = NCHUNK // 2
L = 16

BM = 4096
BK = 4096


def _mv_in_body(w_ref, x_ref, b_ref, o_ref):
    acc = jnp.dot(w_ref[...], x_ref[...], preferred_element_type=jnp.float32)
    o_ref[...] = jnp.maximum(acc + b_ref[...], 0.0)


def _h_matvec(W_in, x, b_in):
    grid = N // BM
    return pl.pallas_call(
        _mv_in_body,
        grid=(grid,),
        in_specs=[
            pl.BlockSpec((BM, IN_DIM), lambda i: (i, 0)),
            pl.BlockSpec((IN_DIM, 1), lambda i: (0, 0)),
            pl.BlockSpec((BM, 1), lambda i: (i, 0)),
        ],
        out_specs=pl.BlockSpec((BM, 1), lambda i: (i, 0)),
        out_shape=jax.ShapeDtypeStruct((N, 1), jnp.float32),
    )(W_in, x.reshape(IN_DIM, 1), b_in.reshape(N, 1))


def _mv_out_body(w_ref, yp_ref, b_ref, o_ref):
    i = pl.program_id(0)
    v = jnp.maximum(jnp.sum(yp_ref[...], axis=0), 0.0).reshape(BK, 1)
    part = jnp.dot(w_ref[...], v, preferred_element_type=jnp.float32)

    @pl.when(i == 0)
    def _():
        o_ref[...] = b_ref[...] + part

    @pl.when(i > 0)
    def _():
        o_ref[...] += part


def _out_matvec(W_out, y_parts, b_out):
    grid = N // BK
    return pl.pallas_call(
        _mv_out_body,
        grid=(grid,),
        in_specs=[
            pl.BlockSpec((OUT_DIM, BK), lambda i: (0, i)),
            pl.BlockSpec((NW, BK), lambda i: (0, i)),
            pl.BlockSpec((OUT_DIM, 1), lambda i: (0, 0)),
        ],
        out_specs=pl.BlockSpec((OUT_DIM, 1), lambda i: (0, 0)),
        out_shape=jax.ShapeDtypeStruct((OUT_DIM, 1), jnp.float32),
    )(W_out, y_parts, b_out.reshape(OUT_DIM, 1))


def _sc_edge_body(h_hbm, cols_hbm, w_hbm, rows_hbm, yp_hbm, c_hbm,
                  hy_v, ia_v, ib_v, va_v, vb_v, ca_v, cb_v,
                  h_sp,
                  sia, sib, sva, svb, sca, scb):
    core = lax.axis_index("c")
    sid = lax.axis_index("s")
    wid = sid * NC + core
    base = wid * EPT

    def start_in(src, ci, buf, sem):
        pltpu.async_copy(src.at[pl.ds(base + ci * CH, CH)], buf, sem)

    def wait_in(src, buf, sem):
        pltpu.make_async_copy(src.at[pl.ds(base, CH)], buf, sem).wait()

    def start_out(buf, ci, sem):
        pltpu.async_copy(buf, c_hbm.at[pl.ds(base + ci * CH, CH)], sem)

    def wait_out(buf, sem):
        pltpu.make_async_copy(buf, c_hbm.at[pl.ds(base, CH)], sem).wait()

    @pl.when(sid == 0)
    def _():
        pltpu.sync_copy(h_hbm, h_sp)

    plsc.subcore_barrier()
    pltpu.sync_copy(h_sp, hy_v)
    start_in(cols_hbm, 0, ia_v, sia)
    start_in(w_hbm, 0, va_v, sva)

    def compute_a(idx_v, w_v, c_v):
        @plsc.parallel_loop(0, CH // L, unroll=8)
        def _(j):
            s = pl.ds(j * L, L)
            c_v[s] = plsc.load_gather(hy_v, [idx_v[s]]) * w_v[s]

    def pair_a(p, _):
        even = 2 * p
        start_in(cols_hbm, even + 1, ib_v, sib)
        start_in(w_hbm, even + 1, vb_v, svb)
        wait_in(cols_hbm, ia_v, sia)
        wait_in(w_hbm, va_v, sva)

        @pl.when(p > 0)
        def _():
            wait_out(ca_v, sca)

        compute_a(ia_v, va_v, ca_v)
        start_out(ca_v, even, sca)

        @pl.when(p < NPAIR - 1)
        def _():
            start_in(cols_hbm, even + 2, ia_v, sia)
            start_in(w_hbm, even + 2, va_v, sva)

        wait_in(cols_hbm, ib_v, sib)
        wait_in(w_hbm, vb_v, svb)

        @pl.when(p > 0)
        def _():
            wait_out(cb_v, scb)

        compute_a(ib_v, vb_v, cb_v)
        start_out(cb_v, even + 1, scb)
        return 0

    lax.fori_loop(0, NPAIR, pair_a, 0)
    wait_out(ca_v, sca)
    wait_out(cb_v, scb)

    zeros = jnp.zeros((L,), jnp.float32)

    @plsc.parallel_loop(0, N // L, unroll=8)
    def _(i):
        hy_v[pl.ds(i * L, L)] = zeros

    start_in(rows_hbm, 0, ia_v, sia)
    start_in(c_hbm, 0, ca_v, sca)

    def compute_b(idx_v, c_v):
        def vec_b(j, _):
            s = pl.ds(j * L, L)
            plsc.addupdate_scatter(hy_v, [idx_v[s]], c_v[s])
            return 0

        lax.fori_loop(0, CH // L, vec_b, 0, unroll=8)

    def pair_b(p, _):
        even = 2 * p
        start_in(rows_hbm, even + 1, ib_v, sib)
        start_in(c_hbm, even + 1, cb_v, scb)
        wait_in(rows_hbm, ia_v, sia)
        wait_in(c_hbm, ca_v, sca)
        compute_b(ia_v, ca_v)

        @pl.when(p < NPAIR - 1)
        def _():
            start_in(rows_hbm, even + 2, ia_v, sia)
            start_in(c_hbm, even + 2, ca_v, sca)

        wait_in(rows_hbm, ib_v, sib)
        wait_in(c_hbm, cb_v, scb)
        compute_b(ib_v, cb_v)
        return 0

    lax.fori_loop(0, NPAIR, pair_b, 0)
    pltpu.sync_copy(hy_v, yp_hbm.at[wid])


_sc_edges = functools.partial(
    pl.kernel,
    out_type=(
        jax.ShapeDtypeStruct((NW, N), jnp.float32),
        jax.ShapeDtypeStruct((NNZ,), jnp.float32),
    ),
    mesh=plsc.VectorSubcoreMesh(
        core_axis_name="c", subcore_axis_name="s",
        num_cores=NC, num_subcores=NS,
    ),
    scratch_types=[
        pltpu.VMEM((N,), jnp.float32),
        pltpu.VMEM((CH,), jnp.int32),
        pltpu.VMEM((CH,), jnp.int32),
        pltpu.VMEM((CH,), jnp.float32),
        pltpu.VMEM((CH,), jnp.float32),
        pltpu.VMEM((CH,), jnp.float32),
        pltpu.VMEM((CH,), jnp.float32),
        pltpu.VMEM_SHARED((N,), jnp.float32),
        pltpu.SemaphoreType.DMA,
        pltpu.SemaphoreType.DMA,
        pltpu.SemaphoreType.DMA,
        pltpu.SemaphoreType.DMA,
        pltpu.SemaphoreType.DMA,
        pltpu.SemaphoreType.DMA,
    ],
    compiler_params=pltpu.CompilerParams(needs_layout_passes=False),
)(_sc_edge_body)


def kernel(x, rows, cols, adj_weights, W_in, b_in, W_out, b_out):
    h = _h_matvec(W_in, x, b_in)
    y_parts, _ = _sc_edges(h.reshape(N), cols, adj_weights, rows)
    out = _out_matvec(W_out, y_parts, b_out)
    return out.reshape(OUT_DIM)


# speedup vs baseline: 1.1294x; 1.0082x over previous
import functools

import jax
import jax.numpy as jnp
from jax import lax
from jax.experimental import pallas as pl
from jax.experimental.pallas import tpu as pltpu
from jax.experimental.pallas import tpu_sc as plsc

N = 65536
NNZ = 4194304
IN_DIM = 512
OUT_DIM = 512

NC = 2
NS = 16
NW = NC * NS
EPT = NNZ // NW
CH = 8192
NCHUNK = EPT // CH
NPAIR = NCHUNK // 2
L = 16

BM = 4096
BK = 4096


def _mv_in_body(w_ref, x_ref, b_ref, o_ref):
    acc = jnp.dot(w_ref[...], x_ref[...], preferred_element_type=jnp.float32)
    o_ref[...] = jnp.maximum(acc + b_ref[...], 0.0)


def _h_matvec(W_in, x, b_in):
    grid = N // BM
    return pl.pallas_call(
        _mv_in_body,
        grid=(grid,),
        in_specs=[
            pl.BlockSpec((BM, IN_DIM), lambda i: (i, 0)),
            pl.BlockSpec((IN_DIM, 1), lambda i: (0, 0)),
            pl.BlockSpec((BM, 1), lambda i: (i, 0)),
        ],
        out_specs=pl.BlockSpec((BM, 1), lambda i: (i, 0)),
        out_shape=jax.ShapeDtypeStruct((N, 1), jnp.float32),
    )(W_in, x.reshape(IN_DIM, 1), b_in.reshape(N, 1))


def _mv_out_body(w_ref, yp_ref, b_ref, o_ref):
    i = pl.program_id(0)
    v = jnp.maximum(jnp.sum(yp_ref[...], axis=0), 0.0).reshape(BK, 1)
    part = jnp.dot(w_ref[...], v, preferred_element_type=jnp.float32)

    @pl.when(i == 0)
    def _():
        o_ref[...] = b_ref[...] + part

    @pl.when(i > 0)
    def _():
        o_ref[...] += part


def _out_matvec(W_out, y_parts, b_out):
    grid = N // BK
    return pl.pallas_call(
        _mv_out_body,
        grid=(grid,),
        in_specs=[
            pl.BlockSpec((OUT_DIM, BK), lambda i: (0, i)),
            pl.BlockSpec((NW, BK), lambda i: (0, i)),
            pl.BlockSpec((OUT_DIM, 1), lambda i: (0, 0)),
        ],
        out_specs=pl.BlockSpec((OUT_DIM, 1), lambda i: (0, 0)),
        out_shape=jax.ShapeDtypeStruct((OUT_DIM, 1), jnp.float32),
    )(W_out, y_parts, b_out.reshape(OUT_DIM, 1))


def _sc_edge_body(h_hbm, cols_hbm, w_hbm, rows_hbm, yp_hbm, c_hbm,
                  hy_v, ia_v, ib_v, va_v, vb_v, ca_v, cb_v,
                  h_sp,
                  sia, sib, sva, svb, sca, scb):
    core = lax.axis_index("c")
    sid = lax.axis_index("s")
    wid = sid * NC + core
    base = wid * EPT

    def start_in(src, ci, buf, sem):
        pltpu.async_copy(src.at[pl.ds(base + ci * CH, CH)], buf, sem)

    def wait_in(src, buf, sem):
        pltpu.make_async_copy(src.at[pl.ds(base, CH)], buf, sem).wait()

    cbase = wid * (EPT // 2)
    CH2 = CH // 2

    def start_cin(ci, buf, sem):
        pltpu.async_copy(c_hbm.at[pl.ds(cbase + ci * CH2, CH2)], buf, sem)

    def wait_cin(buf, sem):
        pltpu.make_async_copy(c_hbm.at[pl.ds(cbase, CH2)], buf, sem).wait()

    def start_out(buf, ci, sem):
        pltpu.async_copy(buf, c_hbm.at[pl.ds(cbase + ci * CH2, CH2)], sem)

    def wait_out(buf, sem):
        pltpu.make_async_copy(buf, c_hbm.at[pl.ds(cbase, CH2)], sem).wait()

    @pl.when(sid == 0)
    def _():
        pltpu.sync_copy(h_hbm, h_sp)

    plsc.subcore_barrier()
    pltpu.sync_copy(h_sp, hy_v)
    start_in(cols_hbm, 0, ia_v, sia)
    start_in(w_hbm, 0, va_v, sva)

    def compute_a(idx_v, w_v, c_v):
        @plsc.parallel_loop(0, CH // (2 * L), unroll=4)
        def _(j):
            s0 = pl.ds(j * 2 * L, L)
            s1 = pl.ds(j * 2 * L + L, L)
            g0 = plsc.load_gather(hy_v, [idx_v[s0]]) * w_v[s0]
            g1 = plsc.load_gather(hy_v, [idx_v[s1]]) * w_v[s1]
            c_v[pl.ds(j * L, L)] = plsc.bitcast(
                plsc.pack(g0, g1, format=plsc.PackFormat.INTERLEAVED),
                jnp.float32)

    def pair_a(p, _):
        even = 2 * p
        start_in(cols_hbm, even + 1, ib_v, sib)
        start_in(w_hbm, even + 1, vb_v, svb)
        wait_in(cols_hbm, ia_v, sia)
        wait_in(w_hbm, va_v, sva)

        @pl.when(p > 0)
        def _():
            wait_out(ca_v, sca)

        compute_a(ia_v, va_v, ca_v)
        start_out(ca_v, even, sca)

        @pl.when(p < NPAIR - 1)
        def _():
            start_in(cols_hbm, even + 2, ia_v, sia)
            start_in(w_hbm, even + 2, va_v, sva)

        wait_in(cols_hbm, ib_v, sib)
        wait_in(w_hbm, vb_v, svb)

        @pl.when(p > 0)
        def _():
            wait_out(cb_v, scb)

        compute_a(ib_v, vb_v, cb_v)
        start_out(cb_v, even + 1, scb)
        return 0

    lax.fori_loop(0, NPAIR, pair_a, 0)
    wait_out(ca_v, sca)
    wait_out(cb_v, scb)

    zeros = jnp.zeros((L,), jnp.float32)

    @plsc.parallel_loop(0, N // L, unroll=8)
    def _(i):
        hy_v[pl.ds(i * L, L)] = zeros

    start_in(rows_hbm, 0, ia_v, sia)
    start_cin(0, ca_v, sca)

    def compute_b(idx_v, c_v):
        def vec_b(j, _):
            cc = plsc.bitcast(c_v[pl.ds(j * L, L)], jnp.bfloat16)
            c0, c1 = plsc.unpack(cc, format=plsc.PackFormat.INTERLEAVED)
            plsc.addupdate_scatter(hy_v, [idx_v[pl.ds(j * 2 * L, L)]], c0)
            plsc.addupdate_scatter(hy_v, [idx_v[pl.ds(j * 2 * L + L, L)]], c1)
            return 0

        lax.fori_loop(0, CH // (2 * L), vec_b, 0, unroll=4)

    def pair_b(p, _):
        even = 2 * p
        start_in(rows_hbm, even + 1, ib_v, sib)
        start_cin(even + 1, cb_v, scb)
        wait_in(rows_hbm, ia_v, sia)
        wait_cin(ca_v, sca)
        compute_b(ia_v, ca_v)

        @pl.when(p < NPAIR - 1)
        def _():
            start_in(rows_hbm, even + 2, ia_v, sia)
            start_cin(even + 2, ca_v, sca)

        wait_in(rows_hbm, ib_v, sib)
        wait_cin(cb_v, scb)
        compute_b(ib_v, cb_v)
        return 0

    lax.fori_loop(0, NPAIR, pair_b, 0)
    pltpu.sync_copy(hy_v, yp_hbm.at[wid])


_sc_edges = functools.partial(
    pl.kernel,
    out_type=(
        jax.ShapeDtypeStruct((NW, N), jnp.float32),
        jax.ShapeDtypeStruct((NNZ // 2,), jnp.float32),
    ),
    mesh=plsc.VectorSubcoreMesh(
        core_axis_name="c", subcore_axis_name="s",
        num_cores=NC, num_subcores=NS,
    ),
    scratch_types=[
        pltpu.VMEM((N,), jnp.float32),
        pltpu.VMEM((CH,), jnp.int32),
        pltpu.VMEM((CH,), jnp.int32),
        pltpu.VMEM((CH,), jnp.float32),
        pltpu.VMEM((CH,), jnp.float32),
        pltpu.VMEM((CH // 2,), jnp.float32),
        pltpu.VMEM((CH // 2,), jnp.float32),
        pltpu.VMEM_SHARED((N,), jnp.float32),
        pltpu.SemaphoreType.DMA,
        pltpu.SemaphoreType.DMA,
        pltpu.SemaphoreType.DMA,
        pltpu.SemaphoreType.DMA,
        pltpu.SemaphoreType.DMA,
        pltpu.SemaphoreType.DMA,
    ],
    compiler_params=pltpu.CompilerParams(needs_layout_passes=False),
)(_sc_edge_body)


def kernel(x, rows, cols, adj_weights, W_in, b_in, W_out, b_out):
    h = _h_matvec(W_in, x, b_in)
    y_parts, _ = _sc_edges(h.reshape(N), cols, adj_weights, rows)
    out = _out_matvec(W_out, y_parts, b_out)
    return out.reshape(OUT_DIM)
